# baseline (speedup 1.0000x reference)
import functools

import jax
import jax.numpy as jnp
from jax import lax
from jax.experimental import pallas as pl
from jax.experimental.pallas import tpu as pltpu
from jax.experimental.pallas import tpu_sc as plsc

_N = 10000
_EMB = 32
_ANCH = 32
_HEADS = 4
_DH = 8
_E0 = 640000
_ADD = int(_E0 * 0.01)
_ETOT = 2 * _ADD + _N + _E0
_L = 16
_NW = 32
_CHUNK = 128
_NCH = -(-_ETOT // (_NW * _CHUNK))
_PER_TILE = _NCH * _CHUNK
_EPAD = _NW * _PER_TILE
_NCHT = _EPAD // _CHUNK
_NPAD = _N + 8
_HPAD = 8
_NBUF = 6


def _qk_body(emb_ref, dst_ref, se_ref, w1_ref, w2_ref, bh_ref, qt_ref,
             kt_ref, q_ref, k_ref):
    f32 = jnp.float32
    sw = jnp.dot(se_ref[...], w1_ref[...], preferred_element_type=f32)
    pos = (jnp.dot(dst_ref[...], sw, preferred_element_type=f32) * (1.0 / _ANCH)
           + jnp.dot(emb_ref[...], w2_ref[...], preferred_element_type=f32)
           + bh_ref[...])
    q_ref[...] = jnp.dot(pos, qt_ref[...], preferred_element_type=f32)
    k_ref[...] = jnp.dot(pos, kt_ref[...], preferred_element_type=f32)


_mesh = plsc.VectorSubcoreMesh(core_axis_name="c", subcore_axis_name="s")
_sc_params = pltpu.CompilerParams(
    needs_layout_passes=False, use_tc_tiling_on_sc=False)


def _edge_attention_body(q_hbm, k_hbm, rc_hbm, z_hbm,
                         exp_hbm, na_hbm, nb_hbm,
                         *refs):
    rcv = refs[0:_NBUF]
    qv = refs[_NBUF:2 * _NBUF]
    kv = refs[2 * _NBUF:3 * _NBUF]
    ev = refs[3 * _NBUF:4 * _NBUF]
    vals = refs[4 * _NBUF:5 * _NBUF]
    rs = refs[5 * _NBUF:6 * _NBUF]
    nsh = refs[6 * _NBUF]
    qsh = refs[6 * _NBUF + 1]
    ksh = refs[6 * _NBUF + 2]
    si = refs[6 * _NBUF + 3:7 * _NBUF + 3]
    sq = refs[7 * _NBUF + 3:8 * _NBUF + 3]
    sk = refs[8 * _NBUF + 3:9 * _NBUF + 3]
    sew = refs[9 * _NBUF + 3:10 * _NBUF + 3]
    sad = refs[10 * _NBUF + 3:11 * _NBUF + 3]
    c = lax.axis_index("c")
    s = lax.axis_index("s")
    wid = s * 2 + c
    base = wid * _PER_TILE
    chbase = wid * _NCH
    for b in range(_NBUF):
        pltpu.sync_copy(z_hbm.at[pl.ds(0, _CHUNK)], vals[b])

    @pl.when(s == 0)
    def _():
        pltpu.sync_copy(z_hbm, nsh)

    @pl.when(s == 1)
    def _():
        pltpu.sync_copy(q_hbm, qsh)

    @pl.when(s == 2)
    def _():
        pltpu.sync_copy(k_hbm, ksh)

    plsc.subcore_barrier()

    def issue_idx(i, b):
        pltpu.async_copy(rc_hbm.at[chbase + i], rcv[b], si[b])

    def wait_idx(i, b):
        pltpu.make_async_copy(rc_hbm.at[chbase + i], rcv[b], si[b]).wait()

    def issue_gather(b):
        pltpu.async_copy(qsh.at[rcv[b].at[0]], qv[b], sq[b])
        pltpu.async_copy(ksh.at[rcv[b].at[1]], kv[b], sk[b])

    def wait_gather(b):
        pltpu.make_async_copy(qsh.at[rcv[b].at[0]], qv[b], sq[b]).wait()
        pltpu.make_async_copy(ksh.at[rcv[b].at[1]], kv[b], sk[b]).wait()

    def wait_write(i, b):
        off = base + i * _CHUNK
        pltpu.make_async_copy(ev[b], exp_hbm.at[pl.ds(off, _CHUNK)], sew[b]).wait()
        pltpu.make_async_copy(vals[b], nsh.at[rs[b]], sad[b]).wait()

    def process(i, b):
        @pl.when(i >= _NBUF)
        def _():
            wait_write(i - _NBUF, b)

        for g in range(_CHUNK // _L):
            ei = lax.iota(jnp.int32, _L) + (g * _L)
            for h in range(_HEADS):
                acc = None
                for d in range(_DH):
                    ci = jnp.full((_L,), h * _DH + d, jnp.int32)
                    qc = plsc.load_gather(qv[b], [ei, ci])
                    kc = plsc.load_gather(kv[b], [ei, ci])
                    acc = qc * kc if acc is None else acc + qc * kc
                att = jnp.minimum(jnp.maximum(acc, -10.0), 10.0)
                ex = jnp.exp(att)
                hs = jnp.full((_L,), h, jnp.int32)
                plsc.store_scatter(vals[b], [ei, hs], ex)
                plsc.store_scatter(ev[b], [ei, hs], ex)
        for j in range(_CHUNK // _L):
            ix = pl.ds(j * _L, _L)
            rs[b][ix] = rcv[b][0, ix]
        off = base + i * _CHUNK
        pltpu.async_copy(ev[b], exp_hbm.at[pl.ds(off, _CHUNK)], sew[b])
        pltpu.async_copy(vals[b], nsh.at[rs[b]], sad[b], add=True)

    for b in range(_NBUF):
        issue_idx(b, b)
    for b in range(_NBUF - 1):
        wait_idx(b, b)
        issue_gather(b)

    def quad_body(k4, carry):
        for b in range(_NBUF):
            i = k4 * _NBUF + b

            @pl.when(i < _NCH)
            def _():
                wait_gather(b)
                process(i, b)

                @pl.when(i + _NBUF < _NCH)
                def _():
                    issue_idx(i + _NBUF, b)

                bn = (b + _NBUF - 1) % _NBUF

                @pl.when(i + _NBUF - 1 < _NCH)
                def _():
                    wait_idx(i + _NBUF - 1, bn)
                    issue_gather(bn)

        return carry

    lax.fori_loop(0, (_NCH + _NBUF - 1) // _NBUF, quad_body, 0)
    for j in range(_NBUF):
        i = _NCH - _NBUF + j
        if i >= 0:
            wait_write(i, i % _NBUF)
    plsc.subcore_barrier()

    @pl.when(jnp.logical_and(s == 0, c == 0))
    def _():
        pltpu.sync_copy(nsh, na_hbm)

    @pl.when(jnp.logical_and(s == 0, c == 1))
    def _():
        pltpu.sync_copy(nsh, nb_hbm)


_edge_attention = functools.partial(
    pl.kernel,
    out_type=[
        jax.ShapeDtypeStruct((_EPAD, _HEADS), jnp.float32),
        jax.ShapeDtypeStruct((_NPAD, _HPAD), jnp.float32),
        jax.ShapeDtypeStruct((_NPAD, _HPAD), jnp.float32),
    ],
    scratch_types=(
        [pltpu.VMEM((2, _CHUNK), jnp.int32)] * _NBUF +
        [pltpu.VMEM((_CHUNK, _EMB), jnp.float32)] * _NBUF +
        [pltpu.VMEM((_CHUNK, _EMB), jnp.float32)] * _NBUF +
        [pltpu.VMEM((_CHUNK, _HEADS), jnp.float32)] * _NBUF +
        [pltpu.VMEM((_CHUNK, _HPAD), jnp.float32)] * _NBUF +
        [pltpu.VMEM((_CHUNK,), jnp.int32)] * _NBUF +
        [pltpu.VMEM_SHARED((_NPAD, _HPAD), jnp.float32)] +
        [pltpu.VMEM_SHARED((_NPAD, _EMB), jnp.float32)] +
        [pltpu.VMEM_SHARED((_NPAD, _EMB), jnp.float32)] +
        [pltpu.SemaphoreType.DMA] * (5 * _NBUF)
    ),
    mesh=_mesh,
    compiler_params=_sc_params,
)(_edge_attention_body)


def _normalize_body(rows_hbm, exp_hbm, na_hbm, nb_hbm, out_hbm, *refs):
    rv = refs[0:_NBUF]
    ev = refs[_NBUF:2 * _NBUF]
    nav = refs[2 * _NBUF:3 * _NBUF]
    nbv = refs[3 * _NBUF:4 * _NBUF]
    av = refs[4 * _NBUF:5 * _NBUF]
    si = refs[5 * _NBUF:6 * _NBUF]
    sa = refs[6 * _NBUF:7 * _NBUF]
    sb = refs[7 * _NBUF:8 * _NBUF]
    sw = refs[8 * _NBUF:9 * _NBUF]
    c = lax.axis_index("c")
    s = lax.axis_index("s")
    wid = s * 2 + c
    base = wid * _PER_TILE

    def issue_idx(i, b):
        off = base + i * _CHUNK
        pltpu.async_copy(rows_hbm.at[pl.ds(off, _CHUNK)], rv[b], si[b])
        pltpu.async_copy(exp_hbm.at[pl.ds(off, _CHUNK)], ev[b], si[b])

    def wait_idx(i, b):
        off = base + i * _CHUNK
        pltpu.make_async_copy(rows_hbm.at[pl.ds(off, _CHUNK)], rv[b], si[b]).wait()
        pltpu.make_async_copy(exp_hbm.at[pl.ds(off, _CHUNK)], ev[b], si[b]).wait()

    def issue_gather(b):
        pltpu.async_copy(na_hbm.at[rv[b]], nav[b], sa[b])
        pltpu.async_copy(nb_hbm.at[rv[b]], nbv[b], sb[b])

    def wait_gather(b):
        pltpu.make_async_copy(na_hbm.at[rv[b]], nav[b], sa[b]).wait()
        pltpu.make_async_copy(nb_hbm.at[rv[b]], nbv[b], sb[b]).wait()

    def wait_write(i, b):
        off = base + i * _CHUNK
        pltpu.make_async_copy(av[b], out_hbm.at[pl.ds(off, _CHUNK)], sw[b]).wait()

    def process(i, b):
        @pl.when(i >= _NBUF)
        def _():
            wait_write(i - _NBUF, b)

        for g in range(_CHUNK // _L):
            ei = lax.iota(jnp.int32, _L) + (g * _L)
            acc = jnp.zeros((_L,), jnp.float32)
            for h in range(_HEADS):
                hs = jnp.full((_L,), h, jnp.int32)
                eh = plsc.load_gather(ev[b], [ei, hs])
                nh = (plsc.load_gather(nav[b], [ei, hs])
                      + plsc.load_gather(nbv[b], [ei, hs]))
                acc = acc + eh / (nh + 1e-8)
            av[b][pl.ds(g * _L, _L)] = acc
        off = base + i * _CHUNK
        pltpu.async_copy(av[b], out_hbm.at[pl.ds(off, _CHUNK)], sw[b])

    for b in range(_NBUF):
        issue_idx(b, b)
    for b in range(_NBUF - 1):
        wait_idx(b, b)
        issue_gather(b)

    def quad_body(k4, carry):
        for b in range(_NBUF):
            i = k4 * _NBUF + b

            @pl.when(i < _NCH)
            def _():
                wait_gather(b)
                process(i, b)

                @pl.when(i + _NBUF < _NCH)
                def _():
                    issue_idx(i + _NBUF, b)

                bn = (b + _NBUF - 1) % _NBUF

                @pl.when(i + _NBUF - 1 < _NCH)
                def _():
                    wait_idx(i + _NBUF - 1, bn)
                    issue_gather(bn)

        return carry

    lax.fori_loop(0, (_NCH + _NBUF - 1) // _NBUF, quad_body, 0)
    for j in range(_NBUF):
        i = _NCH - _NBUF + j
        if i >= 0:
            wait_write(i, i % _NBUF)


_normalize = functools.partial(
    pl.kernel,
    out_type=jax.ShapeDtypeStruct((_EPAD,), jnp.float32),
    scratch_types=(
        [pltpu.VMEM((_CHUNK,), jnp.int32)] * _NBUF +
        [pltpu.VMEM((_CHUNK, _HEADS), jnp.float32)] * _NBUF +
        [pltpu.VMEM((_CHUNK, _HPAD), jnp.float32)] * _NBUF +
        [pltpu.VMEM((_CHUNK, _HPAD), jnp.float32)] * _NBUF +
        [pltpu.VMEM((_CHUNK,), jnp.float32)] * _NBUF +
        [pltpu.SemaphoreType.DMA] * (4 * _NBUF)
    ),
    mesh=_mesh,
    compiler_params=_sc_params,
)(_normalize_body)


def kernel(embeds, edge_index, anchorset_id, dists_array, Wh, bh, qTrans,
           kTrans, vTrans):
    del vTrans
    f32 = jnp.float32
    set_emb = jnp.take(embeds, anchorset_id, axis=0)
    w1 = Wh[:_EMB]
    w2 = Wh[_EMB:]
    emb_p = jnp.pad(embeds, ((0, _NPAD - _N), (0, 0)))
    dst_p = jnp.pad(dists_array, ((0, _NPAD - _N), (0, 0)))
    q_tab, k_tab = pl.pallas_call(
        _qk_body,
        out_shape=[jax.ShapeDtypeStruct((_NPAD, _EMB), f32)] * 2,
    )(emb_p, dst_p, set_emb, w1, w2, bh.reshape(1, _EMB), qTrans, kTrans)

    rows = edge_index[0]
    cols = edge_index[1]
    ka, kb = jax.random.split(jax.random.key(1))
    tr = rows[jax.random.randint(ka, (_ADD,), 0, _E0)]
    tc = cols[jax.random.randint(kb, (_ADD,), 0, _E0)]
    loop = jnp.arange(_N, dtype=rows.dtype)
    new_rows = jnp.concatenate([tr, tc, loop, rows])
    new_cols = jnp.concatenate([tc, tr, loop, cols])
    rows_p = jnp.pad(new_rows, (0, _EPAD - _ETOT), constant_values=_N)
    cols_p = jnp.pad(new_cols, (0, _EPAD - _ETOT), constant_values=_N)
    rc = jnp.stack([rows_p.reshape(_NCHT, _CHUNK),
                    cols_p.reshape(_NCHT, _CHUNK)], axis=1)
    z = jnp.zeros((_NPAD, _HPAD), f32)

    exp_e, na, nb = _edge_attention(q_tab, k_tab, rc, z)
    att = _normalize(rows_p, exp_e, na, nb)
    return att[:_ETOT], new_rows, new_cols


# speedup vs baseline: 1.7027x; 1.7027x over previous
import functools

import jax
import jax.numpy as jnp
from jax import lax
from jax.experimental import pallas as pl
from jax.experimental.pallas import tpu as pltpu
from jax.experimental.pallas import tpu_sc as plsc

_N = 10000
_EMB = 32
_ANCH = 32
_HEADS = 4
_DH = 8
_E0 = 640000
_ADD = int(_E0 * 0.01)
_ETOT = 2 * _ADD + _N + _E0
_L = 16
_NW = 32
_CHUNK = 128
_NCH = -(-_ETOT // (_NW * _CHUNK))
_PER_TILE = _NCH * _CHUNK
_EPAD = _NW * _PER_TILE
_NCHT = _EPAD // _CHUNK
_NPAD = _N + 8
_HPAD = 8
_NBUF = 6


def _qk_body(emb_ref, dst_ref, se_ref, w1_ref, w2_ref, bh_ref, qt_ref,
             kt_ref, q_ref, k_ref):
    f32 = jnp.float32
    sw = jnp.dot(se_ref[...], w1_ref[...], preferred_element_type=f32)
    pos = (jnp.dot(dst_ref[...], sw, preferred_element_type=f32) * (1.0 / _ANCH)
           + jnp.dot(emb_ref[...], w2_ref[...], preferred_element_type=f32)
           + bh_ref[...])
    q_ref[...] = jnp.dot(pos, qt_ref[...], preferred_element_type=f32).astype(
        jnp.bfloat16)
    k_ref[...] = jnp.dot(pos, kt_ref[...], preferred_element_type=f32).astype(
        jnp.bfloat16)


_mesh = plsc.VectorSubcoreMesh(core_axis_name="c", subcore_axis_name="s")
_sc_params = pltpu.CompilerParams(
    needs_layout_passes=False, use_tc_tiling_on_sc=False)


def _edge_attention_body(q_hbm, k_hbm, rc_hbm, z_hbm,
                         exp_hbm, na_hbm, nb_hbm,
                         *refs):
    rcv = refs[0:_NBUF]
    qv = refs[_NBUF:2 * _NBUF]
    kv = refs[2 * _NBUF:3 * _NBUF]
    ev = refs[3 * _NBUF:4 * _NBUF]
    vals = refs[4 * _NBUF:5 * _NBUF]
    rs = refs[5 * _NBUF:6 * _NBUF]
    nsh = refs[6 * _NBUF]
    qsh = refs[6 * _NBUF + 1]
    ksh = refs[6 * _NBUF + 2]
    si = refs[6 * _NBUF + 3:7 * _NBUF + 3]
    sq = refs[7 * _NBUF + 3:8 * _NBUF + 3]
    sk = refs[8 * _NBUF + 3:9 * _NBUF + 3]
    sew = refs[9 * _NBUF + 3:10 * _NBUF + 3]
    sad = refs[10 * _NBUF + 3:11 * _NBUF + 3]
    c = lax.axis_index("c")
    s = lax.axis_index("s")
    wid = s * 2 + c
    base = wid * _PER_TILE
    chbase = wid * _NCH
    for b in range(_NBUF):
        pltpu.sync_copy(z_hbm.at[pl.ds(0, _CHUNK)], vals[b])

    @pl.when(s == 0)
    def _():
        pltpu.sync_copy(z_hbm, nsh)

    @pl.when(s == 1)
    def _():
        pltpu.sync_copy(q_hbm, qsh)

    @pl.when(s == 2)
    def _():
        pltpu.sync_copy(k_hbm, ksh)

    plsc.subcore_barrier()

    def issue_idx(i, b):
        pltpu.async_copy(rc_hbm.at[chbase + i], rcv[b], si[b])

    def wait_idx(i, b):
        pltpu.make_async_copy(rc_hbm.at[chbase + i], rcv[b], si[b]).wait()

    def issue_gather(b):
        pltpu.async_copy(qsh.at[rcv[b].at[0]], qv[b], sq[b])
        pltpu.async_copy(ksh.at[rcv[b].at[1]], kv[b], sk[b])

    def wait_gather(b):
        pltpu.make_async_copy(qsh.at[rcv[b].at[0]], qv[b], sq[b]).wait()
        pltpu.make_async_copy(ksh.at[rcv[b].at[1]], kv[b], sk[b]).wait()

    def wait_write(i, b):
        off = base + i * _CHUNK
        pltpu.make_async_copy(ev[b], exp_hbm.at[pl.ds(off, _CHUNK)], sew[b]).wait()
        pltpu.make_async_copy(vals[b], nsh.at[rs[b]], sad[b]).wait()

    def process(i, b):
        @pl.when(i >= _NBUF)
        def _():
            wait_write(i - _NBUF, b)

        for g in range(_CHUNK // _L):
            ei = lax.iota(jnp.int32, _L) + (g * _L)
            for h in range(_HEADS):
                acc = None
                for d in range(_DH // 2):
                    ci = jnp.full((_L,), h * (_DH // 2) + d, jnp.int32)
                    qw = plsc.load_gather(qv[b], [ei, ci])
                    kw = plsc.load_gather(kv[b], [ei, ci])
                    q0 = plsc.bitcast(qw << 16, jnp.float32)
                    k0 = plsc.bitcast(kw << 16, jnp.float32)
                    q1 = plsc.bitcast((qw >> 16) << 16, jnp.float32)
                    k1 = plsc.bitcast((kw >> 16) << 16, jnp.float32)
                    p = q0 * k0 + q1 * k1
                    acc = p if acc is None else acc + p
                att = jnp.minimum(jnp.maximum(acc, -10.0), 10.0)
                ex = jnp.exp(att)
                hs = jnp.full((_L,), h, jnp.int32)
                plsc.store_scatter(vals[b], [ei, hs], ex)
                plsc.store_scatter(ev[b], [ei, hs], ex)
        for j in range(_CHUNK // _L):
            ix = pl.ds(j * _L, _L)
            rs[b][ix] = rcv[b][0, ix]
        off = base + i * _CHUNK
        pltpu.async_copy(ev[b], exp_hbm.at[pl.ds(off, _CHUNK)], sew[b])
        pltpu.async_copy(vals[b], nsh.at[rs[b]], sad[b], add=True)

    for b in range(_NBUF):
        issue_idx(b, b)
    for b in range(_NBUF - 1):
        wait_idx(b, b)
        issue_gather(b)

    def quad_body(k4, carry):
        for b in range(_NBUF):
            i = k4 * _NBUF + b

            @pl.when(i < _NCH)
            def _():
                wait_gather(b)
                process(i, b)

                @pl.when(i + _NBUF < _NCH)
                def _():
                    issue_idx(i + _NBUF, b)

                bn = (b + _NBUF - 1) % _NBUF

                @pl.when(i + _NBUF - 1 < _NCH)
                def _():
                    wait_idx(i + _NBUF - 1, bn)
                    issue_gather(bn)

        return carry

    lax.fori_loop(0, (_NCH + _NBUF - 1) // _NBUF, quad_body, 0)
    for j in range(_NBUF):
        i = _NCH - _NBUF + j
        if i >= 0:
            wait_write(i, i % _NBUF)
    plsc.subcore_barrier()

    @pl.when(jnp.logical_and(s == 0, c == 0))
    def _():
        pltpu.sync_copy(nsh, na_hbm)

    @pl.when(jnp.logical_and(s == 0, c == 1))
    def _():
        pltpu.sync_copy(nsh, nb_hbm)


_edge_attention = functools.partial(
    pl.kernel,
    out_type=[
        jax.ShapeDtypeStruct((_EPAD, _HEADS), jnp.float32),
        jax.ShapeDtypeStruct((_NPAD, _HPAD), jnp.float32),
        jax.ShapeDtypeStruct((_NPAD, _HPAD), jnp.float32),
    ],
    scratch_types=(
        [pltpu.VMEM((2, _CHUNK), jnp.int32)] * _NBUF +
        [pltpu.VMEM((_CHUNK, _EMB // 2), jnp.int32)] * _NBUF +
        [pltpu.VMEM((_CHUNK, _EMB // 2), jnp.int32)] * _NBUF +
        [pltpu.VMEM((_CHUNK, _HEADS), jnp.float32)] * _NBUF +
        [pltpu.VMEM((_CHUNK, _HPAD), jnp.float32)] * _NBUF +
        [pltpu.VMEM((_CHUNK,), jnp.int32)] * _NBUF +
        [pltpu.VMEM_SHARED((_NPAD, _HPAD), jnp.float32)] +
        [pltpu.VMEM_SHARED((_NPAD, _EMB // 2), jnp.int32)] +
        [pltpu.VMEM_SHARED((_NPAD, _EMB // 2), jnp.int32)] +
        [pltpu.SemaphoreType.DMA] * (5 * _NBUF)
    ),
    mesh=_mesh,
    compiler_params=_sc_params,
)(_edge_attention_body)


def _normalize_body(rows_hbm, exp_hbm, na_hbm, nb_hbm, out_hbm, *refs):
    rv = refs[0:_NBUF]
    ev = refs[_NBUF:2 * _NBUF]
    nav = refs[2 * _NBUF:3 * _NBUF]
    nbv = refs[3 * _NBUF:4 * _NBUF]
    av = refs[4 * _NBUF:5 * _NBUF]
    si = refs[5 * _NBUF:6 * _NBUF]
    sa = refs[6 * _NBUF:7 * _NBUF]
    sb = refs[7 * _NBUF:8 * _NBUF]
    sw = refs[8 * _NBUF:9 * _NBUF]
    c = lax.axis_index("c")
    s = lax.axis_index("s")
    wid = s * 2 + c
    base = wid * _PER_TILE

    def issue_idx(i, b):
        off = base + i * _CHUNK
        pltpu.async_copy(rows_hbm.at[pl.ds(off, _CHUNK)], rv[b], si[b])
        pltpu.async_copy(exp_hbm.at[pl.ds(off, _CHUNK)], ev[b], si[b])

    def wait_idx(i, b):
        off = base + i * _CHUNK
        pltpu.make_async_copy(rows_hbm.at[pl.ds(off, _CHUNK)], rv[b], si[b]).wait()
        pltpu.make_async_copy(exp_hbm.at[pl.ds(off, _CHUNK)], ev[b], si[b]).wait()

    def issue_gather(b):
        pltpu.async_copy(na_hbm.at[rv[b]], nav[b], sa[b])
        pltpu.async_copy(nb_hbm.at[rv[b]], nbv[b], sb[b])

    def wait_gather(b):
        pltpu.make_async_copy(na_hbm.at[rv[b]], nav[b], sa[b]).wait()
        pltpu.make_async_copy(nb_hbm.at[rv[b]], nbv[b], sb[b]).wait()

    def wait_write(i, b):
        off = base + i * _CHUNK
        pltpu.make_async_copy(av[b], out_hbm.at[pl.ds(off, _CHUNK)], sw[b]).wait()

    def process(i, b):
        @pl.when(i >= _NBUF)
        def _():
            wait_write(i - _NBUF, b)

        for g in range(_CHUNK // _L):
            ei = lax.iota(jnp.int32, _L) + (g * _L)
            acc = jnp.zeros((_L,), jnp.float32)
            for h in range(_HEADS):
                hs = jnp.full((_L,), h, jnp.int32)
                eh = plsc.load_gather(ev[b], [ei, hs])
                nh = (plsc.load_gather(nav[b], [ei, hs])
                      + plsc.load_gather(nbv[b], [ei, hs]))
                acc = acc + eh / (nh + 1e-8)
            av[b][pl.ds(g * _L, _L)] = acc
        off = base + i * _CHUNK
        pltpu.async_copy(av[b], out_hbm.at[pl.ds(off, _CHUNK)], sw[b])

    for b in range(_NBUF):
        issue_idx(b, b)
    for b in range(_NBUF - 1):
        wait_idx(b, b)
        issue_gather(b)

    def quad_body(k4, carry):
        for b in range(_NBUF):
            i = k4 * _NBUF + b

            @pl.when(i < _NCH)
            def _():
                wait_gather(b)
                process(i, b)

                @pl.when(i + _NBUF < _NCH)
                def _():
                    issue_idx(i + _NBUF, b)

                bn = (b + _NBUF - 1) % _NBUF

                @pl.when(i + _NBUF - 1 < _NCH)
                def _():
                    wait_idx(i + _NBUF - 1, bn)
                    issue_gather(bn)

        return carry

    lax.fori_loop(0, (_NCH + _NBUF - 1) // _NBUF, quad_body, 0)
    for j in range(_NBUF):
        i = _NCH - _NBUF + j
        if i >= 0:
            wait_write(i, i % _NBUF)


_normalize = functools.partial(
    pl.kernel,
    out_type=jax.ShapeDtypeStruct((_EPAD,), jnp.float32),
    scratch_types=(
        [pltpu.VMEM((_CHUNK,), jnp.int32)] * _NBUF +
        [pltpu.VMEM((_CHUNK, _HEADS), jnp.float32)] * _NBUF +
        [pltpu.VMEM((_CHUNK, _HPAD), jnp.float32)] * _NBUF +
        [pltpu.VMEM((_CHUNK, _HPAD), jnp.float32)] * _NBUF +
        [pltpu.VMEM((_CHUNK,), jnp.float32)] * _NBUF +
        [pltpu.SemaphoreType.DMA] * (4 * _NBUF)
    ),
    mesh=_mesh,
    compiler_params=_sc_params,
)(_normalize_body)


def kernel(embeds, edge_index, anchorset_id, dists_array, Wh, bh, qTrans,
           kTrans, vTrans):
    del vTrans
    f32 = jnp.float32
    set_emb = jnp.take(embeds, anchorset_id, axis=0)
    w1 = Wh[:_EMB]
    w2 = Wh[_EMB:]
    emb_p = jnp.pad(embeds, ((0, _NPAD - _N), (0, 0)))
    dst_p = jnp.pad(dists_array, ((0, _NPAD - _N), (0, 0)))
    q_tab, k_tab = pl.pallas_call(
        _qk_body,
        out_shape=[jax.ShapeDtypeStruct((_NPAD, _EMB), jnp.bfloat16)] * 2,
    )(emb_p, dst_p, set_emb, w1, w2, bh.reshape(1, _EMB), qTrans, kTrans)
    q_i32 = jax.lax.bitcast_convert_type(
        q_tab.reshape(_NPAD, _EMB // 2, 2), jnp.int32)
    k_i32 = jax.lax.bitcast_convert_type(
        k_tab.reshape(_NPAD, _EMB // 2, 2), jnp.int32)

    rows = edge_index[0]
    cols = edge_index[1]
    ka, kb = jax.random.split(jax.random.key(1))
    tr = rows[jax.random.randint(ka, (_ADD,), 0, _E0)]
    tc = cols[jax.random.randint(kb, (_ADD,), 0, _E0)]
    loop = jnp.arange(_N, dtype=rows.dtype)
    new_rows = jnp.concatenate([tr, tc, loop, rows])
    new_cols = jnp.concatenate([tc, tr, loop, cols])
    rows_p = jnp.pad(new_rows, (0, _EPAD - _ETOT), constant_values=_N)
    cols_p = jnp.pad(new_cols, (0, _EPAD - _ETOT), constant_values=_N)
    rc = jnp.stack([rows_p.reshape(_NCHT, _CHUNK),
                    cols_p.reshape(_NCHT, _CHUNK)], axis=1)
    z = jnp.zeros((_NPAD, _HPAD), f32)

    exp_e, na, nb = _edge_attention(q_i32, k_i32, rc, z)
    att = _normalize(rows_p, exp_e, na, nb)
    return att[:_ETOT], new_rows, new_cols


# speedup vs baseline: 1.7088x; 1.0036x over previous
import functools

import jax
import jax.numpy as jnp
from jax import lax
from jax.experimental import pallas as pl
from jax.experimental.pallas import tpu as pltpu
from jax.experimental.pallas import tpu_sc as plsc

_N = 10000
_EMB = 32
_ANCH = 32
_HEADS = 4
_DH = 8
_E0 = 640000
_ADD = int(_E0 * 0.01)
_ETOT = 2 * _ADD + _N + _E0
_L = 16
_NW = 32
_CHUNK = 128
_NCH = -(-_ETOT // (_NW * _CHUNK))
_PER_TILE = _NCH * _CHUNK
_EPAD = _NW * _PER_TILE
_NCHT = _EPAD // _CHUNK
_NPAD = _N + 8
_HPAD = 8
_NBUF = 6


def _qk_body(emb_ref, dst_ref, se_ref, w1_ref, w2_ref, bh_ref, qt_ref,
             kt_ref, q_ref, k_ref):
    f32 = jnp.float32
    sw = jnp.dot(se_ref[...], w1_ref[...], preferred_element_type=f32)
    pos = (jnp.dot(dst_ref[...], sw, preferred_element_type=f32) * (1.0 / _ANCH)
           + jnp.dot(emb_ref[...], w2_ref[...], preferred_element_type=f32)
           + bh_ref[...])
    q_ref[...] = jnp.dot(pos, qt_ref[...], preferred_element_type=f32).astype(
        jnp.bfloat16)
    k_ref[...] = jnp.dot(pos, kt_ref[...], preferred_element_type=f32).astype(
        jnp.bfloat16)


_mesh = plsc.VectorSubcoreMesh(core_axis_name="c", subcore_axis_name="s")
_sc_params = pltpu.CompilerParams(
    needs_layout_passes=False, use_tc_tiling_on_sc=False)


def _edge_attention_body(q_hbm, k_hbm, rc_hbm, z_hbm,
                         exp_hbm, na_hbm, nb_hbm,
                         *refs):
    rcv = refs[0:_NBUF]
    qv = refs[_NBUF:2 * _NBUF]
    kv = refs[2 * _NBUF:3 * _NBUF]
    ev = refs[3 * _NBUF:4 * _NBUF]
    vals = refs[4 * _NBUF:5 * _NBUF]
    rs = refs[5 * _NBUF:6 * _NBUF]
    nsh = refs[6 * _NBUF]
    qsh = refs[6 * _NBUF + 1]
    ksh = refs[6 * _NBUF + 2]
    si = refs[6 * _NBUF + 3:7 * _NBUF + 3]
    sq = refs[7 * _NBUF + 3:8 * _NBUF + 3]
    sk = refs[8 * _NBUF + 3:9 * _NBUF + 3]
    sew = refs[9 * _NBUF + 3:10 * _NBUF + 3]
    sad = refs[10 * _NBUF + 3:11 * _NBUF + 3]
    c = lax.axis_index("c")
    s = lax.axis_index("s")
    wid = s * 2 + c
    base = wid * _PER_TILE
    chbase = wid * _NCH
    for b in range(_NBUF):
        pltpu.sync_copy(z_hbm.at[pl.ds(0, _CHUNK)], vals[b])

    @pl.when(s == 0)
    def _():
        pltpu.sync_copy(z_hbm, nsh)

    @pl.when(s == 1)
    def _():
        pltpu.sync_copy(q_hbm, qsh)

    @pl.when(s == 2)
    def _():
        pltpu.sync_copy(k_hbm, ksh)

    plsc.subcore_barrier()

    def issue_idx(i, b):
        pltpu.async_copy(rc_hbm.at[chbase + i], rcv[b], si[b])

    def wait_idx(i, b):
        pltpu.make_async_copy(rc_hbm.at[chbase + i], rcv[b], si[b]).wait()

    def issue_gather(b):
        pltpu.async_copy(qsh.at[rcv[b].at[0]], qv[b], sq[b])
        pltpu.async_copy(ksh.at[rcv[b].at[1]], kv[b], sk[b])

    def wait_gather(b):
        pltpu.make_async_copy(qsh.at[rcv[b].at[0]], qv[b], sq[b]).wait()
        pltpu.make_async_copy(ksh.at[rcv[b].at[1]], kv[b], sk[b]).wait()

    def wait_write(i, b):
        off = base + i * _CHUNK
        pltpu.make_async_copy(ev[b], exp_hbm.at[pl.ds(off, _CHUNK)], sew[b]).wait()
        pltpu.make_async_copy(vals[b], nsh.at[rs[b]], sad[b]).wait()

    def process(i, b):
        @pl.when(i >= _NBUF)
        def _():
            wait_write(i - _NBUF, b)

        for g in range(_CHUNK // _L):
            ei = lax.iota(jnp.int32, _L) + (g * _L)
            for h in range(_HEADS):
                acc = None
                for d in range(_DH // 2):
                    ci = jnp.full((_L,), h * (_DH // 2) + d, jnp.int32)
                    qw = plsc.load_gather(qv[b], [ei, ci])
                    kw = plsc.load_gather(kv[b], [ei, ci])
                    q0 = plsc.bitcast(qw << 16, jnp.float32)
                    k0 = plsc.bitcast(kw << 16, jnp.float32)
                    q1 = plsc.bitcast((qw >> 16) << 16, jnp.float32)
                    k1 = plsc.bitcast((kw >> 16) << 16, jnp.float32)
                    p = q0 * k0 + q1 * k1
                    acc = p if acc is None else acc + p
                att = jnp.minimum(jnp.maximum(acc, -10.0), 10.0)
                ex = jnp.exp(att)
                hs = jnp.full((_L,), h, jnp.int32)
                plsc.store_scatter(vals[b], [ei, hs], ex)
                plsc.store_scatter(ev[b], [ei, hs], ex)
        for j in range(_CHUNK // _L):
            ix = pl.ds(j * _L, _L)
            rs[b][ix] = rcv[b][0, ix]
        off = base + i * _CHUNK
        pltpu.async_copy(ev[b], exp_hbm.at[pl.ds(off, _CHUNK)], sew[b])
        pltpu.async_copy(vals[b], nsh.at[rs[b]], sad[b], add=True)

    for b in range(_NBUF):
        issue_idx(b, b)
    for b in range(_NBUF - 1):
        wait_idx(b, b)
        issue_gather(b)

    def quad_body(k4, carry):
        for b in range(_NBUF):
            i = k4 * _NBUF + b

            @pl.when(i < _NCH)
            def _():
                wait_gather(b)
                process(i, b)

                @pl.when(i + _NBUF < _NCH)
                def _():
                    issue_idx(i + _NBUF, b)

                bn = (b + _NBUF - 1) % _NBUF

                @pl.when(i + _NBUF - 1 < _NCH)
                def _():
                    wait_idx(i + _NBUF - 1, bn)
                    issue_gather(bn)

        return carry

    lax.fori_loop(0, (_NCH + _NBUF - 1) // _NBUF, quad_body, 0)
    for j in range(_NBUF):
        i = _NCH - _NBUF + j
        if i >= 0:
            wait_write(i, i % _NBUF)
    plsc.subcore_barrier()

    @pl.when(jnp.logical_and(s == 0, c == 0))
    def _():
        pltpu.sync_copy(nsh, na_hbm)

    @pl.when(jnp.logical_and(s == 0, c == 1))
    def _():
        pltpu.sync_copy(nsh, nb_hbm)


_edge_attention = functools.partial(
    pl.kernel,
    out_type=[
        jax.ShapeDtypeStruct((_EPAD, _HEADS), jnp.float32),
        jax.ShapeDtypeStruct((_NPAD, _HPAD), jnp.float32),
        jax.ShapeDtypeStruct((_NPAD, _HPAD), jnp.float32),
    ],
    scratch_types=(
        [pltpu.VMEM((2, _CHUNK), jnp.int32)] * _NBUF +
        [pltpu.VMEM((_CHUNK, _EMB // 2), jnp.int32)] * _NBUF +
        [pltpu.VMEM((_CHUNK, _EMB // 2), jnp.int32)] * _NBUF +
        [pltpu.VMEM((_CHUNK, _HEADS), jnp.float32)] * _NBUF +
        [pltpu.VMEM((_CHUNK, _HPAD), jnp.float32)] * _NBUF +
        [pltpu.VMEM((_CHUNK,), jnp.int32)] * _NBUF +
        [pltpu.VMEM_SHARED((_NPAD, _HPAD), jnp.float32)] +
        [pltpu.VMEM_SHARED((_NPAD, _EMB // 2), jnp.int32)] +
        [pltpu.VMEM_SHARED((_NPAD, _EMB // 2), jnp.int32)] +
        [pltpu.SemaphoreType.DMA] * (5 * _NBUF)
    ),
    mesh=_mesh,
    compiler_params=_sc_params,
)(_edge_attention_body)


def _normalize_body(rc_hbm, exp_hbm, na_hbm, nb_hbm, out_hbm, *refs):
    rv = refs[0:_NBUF]
    ev = refs[_NBUF:2 * _NBUF]
    nav = refs[2 * _NBUF:3 * _NBUF]
    nbv = refs[3 * _NBUF:4 * _NBUF]
    av = refs[4 * _NBUF:5 * _NBUF]
    si = refs[5 * _NBUF:6 * _NBUF]
    sa = refs[6 * _NBUF:7 * _NBUF]
    sb = refs[7 * _NBUF:8 * _NBUF]
    sw = refs[8 * _NBUF:9 * _NBUF]
    c = lax.axis_index("c")
    s = lax.axis_index("s")
    wid = s * 2 + c
    base = wid * _PER_TILE
    chbase = wid * _NCH

    def issue_idx(i, b):
        off = base + i * _CHUNK
        pltpu.async_copy(rc_hbm.at[chbase + i, 0], rv[b], si[b])
        pltpu.async_copy(exp_hbm.at[pl.ds(off, _CHUNK)], ev[b], si[b])

    def wait_idx(i, b):
        off = base + i * _CHUNK
        pltpu.make_async_copy(rc_hbm.at[chbase + i, 0], rv[b], si[b]).wait()
        pltpu.make_async_copy(exp_hbm.at[pl.ds(off, _CHUNK)], ev[b], si[b]).wait()

    def issue_gather(b):
        pltpu.async_copy(na_hbm.at[rv[b]], nav[b], sa[b])
        pltpu.async_copy(nb_hbm.at[rv[b]], nbv[b], sb[b])

    def wait_gather(b):
        pltpu.make_async_copy(na_hbm.at[rv[b]], nav[b], sa[b]).wait()
        pltpu.make_async_copy(nb_hbm.at[rv[b]], nbv[b], sb[b]).wait()

    def wait_write(i, b):
        off = base + i * _CHUNK
        pltpu.make_async_copy(av[b], out_hbm.at[pl.ds(off, _CHUNK)], sw[b]).wait()

    def process(i, b):
        @pl.when(i >= _NBUF)
        def _():
            wait_write(i - _NBUF, b)

        for g in range(_CHUNK // _L):
            ei = lax.iota(jnp.int32, _L) + (g * _L)
            acc = jnp.zeros((_L,), jnp.float32)
            for h in range(_HEADS):
                hs = jnp.full((_L,), h, jnp.int32)
                eh = plsc.load_gather(ev[b], [ei, hs])
                nh = (plsc.load_gather(nav[b], [ei, hs])
                      + plsc.load_gather(nbv[b], [ei, hs]))
                acc = acc + eh / (nh + 1e-8)
            av[b][pl.ds(g * _L, _L)] = acc
        off = base + i * _CHUNK
        pltpu.async_copy(av[b], out_hbm.at[pl.ds(off, _CHUNK)], sw[b])

    for b in range(_NBUF):
        issue_idx(b, b)
    for b in range(_NBUF - 1):
        wait_idx(b, b)
        issue_gather(b)

    def quad_body(k4, carry):
        for b in range(_NBUF):
            i = k4 * _NBUF + b

            @pl.when(i < _NCH)
            def _():
                wait_gather(b)
                process(i, b)

                @pl.when(i + _NBUF < _NCH)
                def _():
                    issue_idx(i + _NBUF, b)

                bn = (b + _NBUF - 1) % _NBUF

                @pl.when(i + _NBUF - 1 < _NCH)
                def _():
                    wait_idx(i + _NBUF - 1, bn)
                    issue_gather(bn)

        return carry

    lax.fori_loop(0, (_NCH + _NBUF - 1) // _NBUF, quad_body, 0)
    for j in range(_NBUF):
        i = _NCH - _NBUF + j
        if i >= 0:
            wait_write(i, i % _NBUF)


_normalize = functools.partial(
    pl.kernel,
    out_type=jax.ShapeDtypeStruct((_EPAD,), jnp.float32),
    scratch_types=(
        [pltpu.VMEM((_CHUNK,), jnp.int32)] * _NBUF +
        [pltpu.VMEM((_CHUNK, _HEADS), jnp.float32)] * _NBUF +
        [pltpu.VMEM((_CHUNK, _HPAD), jnp.float32)] * _NBUF +
        [pltpu.VMEM((_CHUNK, _HPAD), jnp.float32)] * _NBUF +
        [pltpu.VMEM((_CHUNK,), jnp.float32)] * _NBUF +
        [pltpu.SemaphoreType.DMA] * (4 * _NBUF)
    ),
    mesh=_mesh,
    compiler_params=_sc_params,
)(_normalize_body)


def kernel(embeds, edge_index, anchorset_id, dists_array, Wh, bh, qTrans,
           kTrans, vTrans):
    del vTrans
    f32 = jnp.float32
    set_emb = jnp.take(embeds, anchorset_id, axis=0)
    w1 = Wh[:_EMB]
    w2 = Wh[_EMB:]
    emb_p = jnp.pad(embeds, ((0, _NPAD - _N), (0, 0)))
    dst_p = jnp.pad(dists_array, ((0, _NPAD - _N), (0, 0)))
    q_tab, k_tab = pl.pallas_call(
        _qk_body,
        out_shape=[jax.ShapeDtypeStruct((_NPAD, _EMB), jnp.bfloat16)] * 2,
    )(emb_p, dst_p, set_emb, w1, w2, bh.reshape(1, _EMB), qTrans, kTrans)
    q_i32 = jax.lax.bitcast_convert_type(
        q_tab.reshape(_NPAD, _EMB // 2, 2), jnp.int32)
    k_i32 = jax.lax.bitcast_convert_type(
        k_tab.reshape(_NPAD, _EMB // 2, 2), jnp.int32)

    rows = edge_index[0]
    cols = edge_index[1]
    ka, kb = jax.random.split(jax.random.key(1))
    tr = rows[jax.random.randint(ka, (_ADD,), 0, _E0)]
    tc = cols[jax.random.randint(kb, (_ADD,), 0, _E0)]
    loop = jnp.arange(_N, dtype=rows.dtype)
    new_rows = jnp.concatenate([tr, tc, loop, rows])
    new_cols = jnp.concatenate([tc, tr, loop, cols])
    rows_p = jnp.pad(new_rows, (0, _EPAD - _ETOT), constant_values=_N)
    cols_p = jnp.pad(new_cols, (0, _EPAD - _ETOT), constant_values=_N)
    rc = jnp.stack([rows_p.reshape(_NCHT, _CHUNK),
                    cols_p.reshape(_NCHT, _CHUNK)], axis=1)
    z = jnp.zeros((_NPAD, _HPAD), f32)

    exp_e, na, nb = _edge_attention(q_i32, k_i32, rc, z)
    att = _normalize(rc, exp_e, na, nb)
    return att[:_ETOT], new_rows, new_cols
